# Initial kernel scaffold; baseline (speedup 1.0000x reference)
import functools

import jax
import jax.numpy as jnp
from jax import lax
from jax.experimental import pallas as pl
from jax.experimental.pallas import tpu as pltpu
from jax.experimental.pallas import tpu_sc as plsc

_N = 10000
_E = 320000
_D = 128
_L = 4
_G = 400
_NA = 9
_AV = 64
_NB = 3
_BV = 8

_NC = 2
_NS = 16
_NW = _NC * _NS
_EW = _E // _NW

_AB = 400
_ABLK = _EW // _AB
_B = 80
_NBLK = _EW // _B

_R = 1000
_NRB = _N // _R


def _sc_mesh():
    return plsc.VectorSubcoreMesh(core_axis_name="c", subcore_axis_name="s")


def _sc_prep_body(ei_hbm, ef_hbm, deg_out, src_out, dst_out, fus_out,
                  srcb, dstb, efb, fusb, ones80, zbuf, deg_acc, ssem):
    cid = lax.axis_index("c")
    sid = lax.axis_index("s")
    wid = sid * _NC + cid

    def _z(k, _):
        zbuf[pl.ds(k * 16, 16)] = jnp.zeros((16,), jnp.float32)
        return 0
    lax.fori_loop(0, 40, _z, 0)
    for k in range(5):
        ones80[pl.ds(k * 16, 16)] = jnp.full((16,), 1.0, jnp.float32)

    @pl.when(sid < 15)
    def _():
        pltpu.sync_copy(zbuf, deg_acc.at[pl.ds(sid * 640, 640)])

    @pl.when(sid == 15)
    def _():
        pltpu.sync_copy(zbuf.at[pl.ds(0, 400)], deg_acc.at[pl.ds(9600, 400)])

    plsc.subcore_barrier()

    def blk(j, _):
        base = wid * _EW + j * _AB
        for r in range(5):
            pltpu.sync_copy(ei_hbm.at[0, pl.ds(base + r * 80, 80)], srcb.at[r])
            pltpu.sync_copy(ei_hbm.at[1, pl.ds(base + r * 80, 80)], dstb.at[r])
        pltpu.sync_copy(ef_hbm.at[pl.ds(base, _AB), :], efb)

        for r in range(5):
            for k in range(5):
                rows = lax.iota(jnp.int32, 16) + (r * 80 + k * 16)
                c0 = plsc.load_gather(efb, [rows, jnp.full((16,), 0, jnp.int32)])
                c1 = plsc.load_gather(efb, [rows, jnp.full((16,), 1, jnp.int32)])
                c2 = plsc.load_gather(efb, [rows, jnp.full((16,), 2, jnp.int32)])
                fusb[r, pl.ds(k * 16, 16)] = c0 * 64 + c1 * 8 + c2

        for r in range(5):
            pltpu.async_copy(ones80, deg_acc.at[dstb.at[r]], ssem, add=True)
        for r in range(5):
            pltpu.make_async_copy(ones80, deg_acc.at[dstb.at[0]], ssem).wait()

        pltpu.sync_copy(srcb, src_out.at[wid, pl.ds(j * 5, 5), :])
        pltpu.sync_copy(dstb, dst_out.at[wid, pl.ds(j * 5, 5), :])
        pltpu.sync_copy(fusb, fus_out.at[wid, pl.ds(j * 5, 5), :])
        return 0

    lax.fori_loop(0, _ABLK, blk, 0)
    plsc.subcore_barrier()

    @pl.when(sid < 15)
    def _():
        pltpu.sync_copy(deg_acc.at[pl.ds(sid * 640, 640)],
                        deg_out.at[cid, pl.ds(sid * 640, 640)])

    @pl.when(sid == 15)
    def _():
        pltpu.sync_copy(deg_acc.at[pl.ds(9600, 400)],
                        deg_out.at[cid, pl.ds(9600, 400)])


def _sc_prep(edge_index, efeat):
    f = pl.kernel(
        _sc_prep_body,
        out_type=[
            jax.ShapeDtypeStruct((_NC, _N), jnp.float32),
            jax.ShapeDtypeStruct((_NW, _NBLK, _B), jnp.int32),
            jax.ShapeDtypeStruct((_NW, _NBLK, _B), jnp.int32),
            jax.ShapeDtypeStruct((_NW, _NBLK, _B), jnp.int32),
        ],
        mesh=_sc_mesh(),
        scratch_types=[
            pltpu.VMEM((5, 80), jnp.int32),
            pltpu.VMEM((5, 80), jnp.int32),
            pltpu.VMEM((_AB, 3), jnp.int32),
            pltpu.VMEM((5, 80), jnp.int32),
            pltpu.VMEM((80,), jnp.float32),
            pltpu.VMEM((640,), jnp.float32),
            pltpu.VMEM_SHARED((_N,), jnp.float32),
            pltpu.SemaphoreType.DMA,
        ],
    )
    return f(edge_index, efeat)


def _tc_prep_kernel(nfeat_ref, atab_ref, degp_ref, btab_ref, w0_ref, b0_ref,
                    hp0_ref, norm_ref, rdeg_ref, ftab_ref):
    i = pl.program_id(0)
    nf = nfeat_ref[...]
    iota_av = lax.broadcasted_iota(jnp.int32, (1, _AV), 1)
    h0 = jnp.zeros((_R, _D), jnp.float32)
    for f in range(_NA):
        oh = (nf[:, f:f + 1] == iota_av).astype(jnp.float32)
        h0 = h0 + jnp.dot(oh, atab_ref[f], preferred_element_type=jnp.float32)

    dg = degp_ref[0:1, :] + degp_ref[1:2, :] + 1.0
    norm_ref[...] = lax.rsqrt(dg)
    rdeg_ref[...] = 1.0 / dg
    hp0_ref[...] = jnp.dot(h0, w0_ref[...],
                           preferred_element_type=jnp.float32) + b0_ref[...]

    @pl.when(i == 0)
    def _():
        i512 = lax.broadcasted_iota(jnp.int32, (512, 1), 0)
        iota_bv = lax.broadcasted_iota(jnp.int32, (1, _BV), 1)
        sel0 = (i512 // 64 == iota_bv).astype(jnp.float32)
        sel1 = ((i512 // 8) % 8 == iota_bv).astype(jnp.float32)
        sel2 = (i512 % 8 == iota_bv).astype(jnp.float32)
        for l in range(_L):
            ft = (jnp.dot(sel0, btab_ref[l, 0], preferred_element_type=jnp.float32)
                  + jnp.dot(sel1, btab_ref[l, 1], preferred_element_type=jnp.float32)
                  + jnp.dot(sel2, btab_ref[l, 2], preferred_element_type=jnp.float32))
            ftab_ref[l] = ft


def _tc_prep(nfeat, atom_tables, deg_parts, bond_tables, W0, b0):
    return pl.pallas_call(
        _tc_prep_kernel,
        grid=(_NRB,),
        in_specs=[
            pl.BlockSpec((_R, _NA), lambda i: (i, 0)),
            pl.BlockSpec((_NA, _AV, _D), lambda i: (0, 0, 0)),
            pl.BlockSpec((_NC, _R), lambda i: (0, i)),
            pl.BlockSpec((_L, _NB, _BV, _D), lambda i: (0, 0, 0, 0)),
            pl.BlockSpec((_D, _D), lambda i: (0, 0)),
            pl.BlockSpec((1, _D), lambda i: (0, 0)),
        ],
        out_specs=[
            pl.BlockSpec((_R, _D), lambda i: (i, 0)),
            pl.BlockSpec((1, _R), lambda i: (0, i)),
            pl.BlockSpec((1, _R), lambda i: (0, i)),
            pl.BlockSpec((_L, 512, _D), lambda i: (0, 0, 0)),
        ],
        out_shape=[
            jax.ShapeDtypeStruct((_N, _D), jnp.float32),
            jax.ShapeDtypeStruct((1, _N), jnp.float32),
            jax.ShapeDtypeStruct((1, _N), jnp.float32),
            jax.ShapeDtypeStruct((_L, 512, _D), jnp.float32),
        ],
    )(nfeat, atom_tables, deg_parts, bond_tables, W0, b0)


def _sc_edge_body(hp_hbm, src_hbm, dst_hbm, fus_hbm, norm_hbm, ftab_hbm,
                  agg_out,
                  ftab_v, src2, dst2, fus2, norm_v, nebuf, buf, agg_acc,
                  gsem0, gsem1, ssem0, ssem1):
    cid = lax.axis_index("c")
    sid = lax.axis_index("s")
    wid = sid * _NC + cid
    gsems = (gsem0, gsem1)
    ssems = (ssem0, ssem1)

    pltpu.sync_copy(src_hbm.at[wid], src2)
    pltpu.sync_copy(dst_hbm.at[wid], dst2)
    pltpu.sync_copy(fus_hbm.at[wid], fus2)
    pltpu.sync_copy(norm_hbm, norm_v)
    pltpu.sync_copy(ftab_hbm, ftab_v)

    def _z(k, _):
        for c in range(8):
            buf[0, k, pl.ds(c * 16, 16)] = jnp.zeros((16,), jnp.float32)
        return 0
    lax.fori_loop(0, _B, _z, 0)
    r0 = sid * 625
    for k in range(7):
        pltpu.sync_copy(buf.at[0], agg_acc.at[pl.ds(r0 + k * 80, 80), :])
    pltpu.sync_copy(buf.at[0, pl.ds(0, 65)],
                    agg_acc.at[pl.ds(r0 + 560, 65), :])
    plsc.subcore_barrier()

    def start_gather(j, s):
        pltpu.async_copy(hp_hbm.at[src2.at[j]], buf.at[s], gsems[s])

    def wait_gather(s):
        pltpu.make_async_copy(hp_hbm.at[src2.at[0]], buf.at[s],
                              gsems[s]).wait()

    def start_scatter(j, s):
        pltpu.async_copy(buf.at[s], agg_acc.at[dst2.at[j]], ssems[s],
                         add=True)

    def wait_scatter(s):
        pltpu.make_async_copy(buf.at[s], agg_acc.at[dst2.at[0]],
                              ssems[s]).wait()

    def compute(j, s):
        for k in range(5):
            sv = src2[j, pl.ds(k * 16, 16)]
            dv = dst2[j, pl.ds(k * 16, 16)]
            ns = plsc.load_gather(norm_v, [sv])
            nd = plsc.load_gather(norm_v, [dv])
            nebuf[pl.ds(k * 16, 16)] = ns * nd

        def ebody(i, _):
            ne = nebuf[i]
            fi = fus2[j, i]
            nev = jnp.full((16,), ne, jnp.float32)
            for c in range(8):
                v = buf[s, i, pl.ds(c * 16, 16)]
                t = ftab_v[fi, pl.ds(c * 16, 16)]
                buf[s, i, pl.ds(c * 16, 16)] = jnp.maximum(v + t, 0.0) * nev
            return 0
        lax.fori_loop(0, _B, ebody, 0)

    start_gather(0, 0)
    start_gather(1, 1)

    def pair(p, _):
        for s in range(2):
            j = 2 * p + s
            wait_gather(s)
            compute(j, s)
            start_scatter(j, s)
            nj = j + 2

            @pl.when(nj < _NBLK)
            def _():
                wait_scatter(s)
                start_gather(nj, s)
        return 0

    lax.fori_loop(0, (_NBLK - 1) // 2, pair, 0)
    jt = _NBLK - 1
    wait_gather(0)
    compute(jt, 0)
    start_scatter(jt, 0)
    wait_scatter(0)
    wait_scatter(1)
    plsc.subcore_barrier()

    pltpu.sync_copy(agg_acc.at[pl.ds(r0, 625), :],
                    agg_out.at[cid, pl.ds(r0, 625), :])


def _sc_edge_make():
    return pl.kernel(
        _sc_edge_body,
        out_type=[jax.ShapeDtypeStruct((_NC, _N, _D), jnp.float32)],
        mesh=_sc_mesh(),
        scratch_types=[
            pltpu.VMEM((512, _D), jnp.float32),
            pltpu.VMEM((_NBLK, _B), jnp.int32),
            pltpu.VMEM((_NBLK, _B), jnp.int32),
            pltpu.VMEM((_NBLK, _B), jnp.int32),
            pltpu.VMEM((_N,), jnp.float32),
            pltpu.VMEM((_B,), jnp.float32),
            pltpu.VMEM((2, _B, _D), jnp.float32),
            pltpu.VMEM_SHARED((_N, _D), jnp.float32),
            pltpu.SemaphoreType.DMA,
            pltpu.SemaphoreType.DMA,
            pltpu.SemaphoreType.DMA,
            pltpu.SemaphoreType.DMA,
        ],
    )


def _tc_dense_kernel(last, agg0_ref, agg1_ref, hp_ref, rdeg_ref, res_ref,
                     gam_ref, bet_ref, wn_ref, bn_ref,
                     hpre_ref, out_ref, stats):
    p = pl.program_id(0)
    i = pl.program_id(1)

    @pl.when(p == 0)
    def _():
        resid = jnp.maximum(hp_ref[...] + res_ref[...], 0.0) * rdeg_ref[...]
        pre = agg0_ref[...] + agg1_ref[...] + resid
        hpre_ref[...] = pre
        s = jnp.sum(pre, axis=0, keepdims=True)
        sq = jnp.sum(pre * pre, axis=0, keepdims=True)

        @pl.when(i == 0)
        def _():
            stats[0:1, :] = s
            stats[1:2, :] = sq

        @pl.when(i > 0)
        def _():
            stats[0:1, :] = stats[0:1, :] + s
            stats[1:2, :] = stats[1:2, :] + sq

    @pl.when(p == 1)
    def _():
        mean = stats[0:1, :] / float(_N)
        var = stats[1:2, :] / float(_N) - mean * mean
        inv = lax.rsqrt(var + 1e-5)
        h = jnp.maximum((hpre_ref[...] - mean) * inv * gam_ref[...]
                        + bet_ref[...], 0.0)
        if last:
            out_ref[...] = h
        else:
            out_ref[...] = jnp.dot(h, wn_ref[...],
                                   preferred_element_type=jnp.float32) + bn_ref[...]


def _tc_dense(agg0, agg1, hp, rdeg_c, res_l, gam_l, bet_l, wn, bn, last):
    return pl.pallas_call(
        functools.partial(_tc_dense_kernel, last),
        grid=(2, _NRB),
        in_specs=[
            pl.BlockSpec((_R, _D), lambda p, i: (i, 0)),
            pl.BlockSpec((_R, _D), lambda p, i: (i, 0)),
            pl.BlockSpec((_R, _D), lambda p, i: (i, 0)),
            pl.BlockSpec((_R, 1), lambda p, i: (i, 0)),
            pl.BlockSpec((1, _D), lambda p, i: (0, 0)),
            pl.BlockSpec((1, _D), lambda p, i: (0, 0)),
            pl.BlockSpec((1, _D), lambda p, i: (0, 0)),
            pl.BlockSpec((_D, _D), lambda p, i: (0, 0)),
            pl.BlockSpec((1, _D), lambda p, i: (0, 0)),
        ],
        out_specs=[
            pl.BlockSpec((_R, _D), lambda p, i: (i, 0)),
            pl.BlockSpec((_R, _D), lambda p, i: (i, 0)),
        ],
        out_shape=[
            jax.ShapeDtypeStruct((_N, _D), jnp.float32),
            jax.ShapeDtypeStruct((_N, _D), jnp.float32),
        ],
        scratch_shapes=[pltpu.VMEM((8, _D), jnp.float32)],
    )(agg0, agg1, hp, rdeg_c, res_l, gam_l, bet_l, wn, bn)[1]


def _tc_pool_kernel(h_ref, gid_ref, w1_ref, b1_ref, w2_ref, b2_ref,
                    out_ref, gsum, cnt):
    i = pl.program_id(0)
    ids = gid_ref[...]
    iota_g = lax.broadcasted_iota(jnp.int32, (_G, _R), 0)
    oht = (iota_g == ids).astype(jnp.float32)
    part = jnp.dot(oht, h_ref[...], preferred_element_type=jnp.float32)
    c = jnp.sum(oht, axis=1, keepdims=True)

    @pl.when(i == 0)
    def _():
        gsum[...] = part
        cnt[:, 0:1] = c

    @pl.when(i > 0)
    def _():
        gsum[...] = gsum[...] + part
        cnt[:, 0:1] = cnt[:, 0:1] + c

    @pl.when(i == _NRB - 1)
    def _():
        g = gsum[...] / jnp.maximum(cnt[:, 0:1], 1.0)
        z = jnp.maximum(jnp.dot(g, w1_ref[...],
                                preferred_element_type=jnp.float32)
                        + b1_ref[...], 0.0)
        out_ref[...] = jnp.dot(z, w2_ref[...],
                               preferred_element_type=jnp.float32) + b2_ref[...]


def _tc_pool(h, gid2d, W1, b1, W2, b2):
    return pl.pallas_call(
        _tc_pool_kernel,
        grid=(_NRB,),
        in_specs=[
            pl.BlockSpec((_R, _D), lambda i: (i, 0)),
            pl.BlockSpec((1, _R), lambda i: (0, i)),
            pl.BlockSpec((_D, _D // 2), lambda i: (0, 0)),
            pl.BlockSpec((1, _D // 2), lambda i: (0, 0)),
            pl.BlockSpec((_D // 2, 1), lambda i: (0, 0)),
            pl.BlockSpec((1, 1), lambda i: (0, 0)),
        ],
        out_specs=pl.BlockSpec((_G, 1), lambda i: (0, 0)),
        out_shape=jax.ShapeDtypeStruct((_G, 1), jnp.float32),
        scratch_shapes=[
            pltpu.VMEM((_G, _D), jnp.float32),
            pltpu.VMEM((_G, 8), jnp.float32),
        ],
    )(h, gid2d, W1, b1, W2, b2)


def kernel(nfeat, efeat, edge_index, node_graph_ids, atom_tables, bond_tables,
           W, b, res_w, gamma, beta, W1, b1, W2, b2):
    deg_parts, src_r, dst_r, fus_r = _sc_prep(edge_index, efeat)

    hp, norm2, rdeg2, ftabs = _tc_prep(
        nfeat, atom_tables, deg_parts, bond_tables,
        W[0], b[0].reshape(1, _D))

    norm1 = norm2.reshape(_N)
    rdeg_c = rdeg2.reshape(_N, 1)

    sc_edge = _sc_edge_make()
    for l in range(_L):
        (aggp,) = sc_edge(hp, src_r, dst_r, fus_r, norm1, ftabs[l])
        last = l == _L - 1
        wn = W[l] if last else W[l + 1]
        bn = (b[l] if last else b[l + 1]).reshape(1, _D)
        hp = _tc_dense(aggp[0], aggp[1], hp, rdeg_c,
                       res_w[l], gamma[l].reshape(1, _D),
                       beta[l].reshape(1, _D), wn, bn, last)

    return _tc_pool(hp, node_graph_ids.reshape(1, _N),
                    W1, b1.reshape(1, _D // 2), W2, b2.reshape(1, 1))


# speedup vs baseline: 11.3036x; 11.3036x over previous
import functools

import jax
import jax.numpy as jnp
from jax import lax
from jax.experimental import pallas as pl
from jax.experimental.pallas import tpu as pltpu
from jax.experimental.pallas import tpu_sc as plsc

_N = 10000
_E = 320000
_D = 128
_L = 4
_G = 400
_NA = 9
_AV = 64
_NB = 3
_BV = 8

_NC = 2
_NS = 16
_NW = _NC * _NS
_EW = _E // _NW

_AB = 640
_ABLK = 15
_B = 16
_NBLK = _EW // _B
_MW = 48
_MWK = _NBLK * _MW

_R = 1000
_NRB = _N // _R
_NP = 10240


def _sc_mesh():
    return plsc.VectorSubcoreMesh(core_axis_name="c", subcore_axis_name="s")


def _sc_prep_body(srcf_hbm, dstf_hbm, ef_hbm, deg_out, meta_out,
                  srcb1, dstb1, dst_sc, efb, mblk, ones80, zbuf, deg_acc, ssem):
    cid = lax.axis_index("c")
    sid = lax.axis_index("s")
    wid = sid * _NC + cid

    def _z(k, _):
        zbuf[pl.ds(k * 16, 16)] = jnp.zeros((16,), jnp.float32)
        return 0
    lax.fori_loop(0, 40, _z, 0)
    for k in range(5):
        ones80[pl.ds(k * 16, 16)] = jnp.full((16,), 1.0, jnp.float32)

    pltpu.sync_copy(zbuf, deg_acc.at[pl.ds(sid * 640, 640)])
    plsc.subcore_barrier()

    def process(base, nsb):
        n = nsb * 16
        pltpu.sync_copy(srcf_hbm.at[pl.ds(base, n)], srcb1.at[pl.ds(0, n)])
        pltpu.sync_copy(dstf_hbm.at[pl.ds(base, n)], dstb1.at[pl.ds(0, n)])
        pltpu.sync_copy(ef_hbm.at[pl.ds(base * 3, n * 3)],
                        efb.at[pl.ds(0, n * 3)])

        def sub(t, _):
            sv = srcb1[pl.ds(t * 16, 16)]
            dv = dstb1[pl.ds(t * 16, 16)]
            rows = (lax.iota(jnp.int32, 16) + t * 16) * 3
            c0 = plsc.load_gather(efb, [rows])
            c1 = plsc.load_gather(efb, [rows + 1])
            c2 = plsc.load_gather(efb, [rows + 2])
            mblk[pl.ds(t * _MW, 16)] = sv
            mblk[pl.ds(t * _MW + 16, 16)] = dv
            mblk[pl.ds(t * _MW + 32, 16)] = c0 * 64 + c1 * 8 + c2
            return 0
        lax.fori_loop(0, nsb, sub, 0)

        nrows = n // 80
        for r in range(8):
            if r < nrows:
                for k in range(5):
                    dst_sc[r, pl.ds(k * 16, 16)] = dstb1[pl.ds(r * 80 + k * 16, 16)]
        for r in range(nrows):
            pltpu.async_copy(ones80, deg_acc.at[dst_sc.at[r]], ssem, add=True)
        for r in range(nrows):
            pltpu.make_async_copy(ones80, deg_acc.at[dst_sc.at[0]], ssem).wait()

    def blk(j, _):
        process(wid * _EW + j * _AB, 40)
        pltpu.sync_copy(mblk, meta_out.at[pl.ds(wid * _MWK + j * 1920, 1920)])
        return 0

    lax.fori_loop(0, _ABLK, blk, 0)
    process(wid * _EW + _ABLK * _AB, 25)
    pltpu.sync_copy(mblk.at[pl.ds(0, 1200)],
                    meta_out.at[pl.ds(wid * _MWK + _ABLK * 1920, 1200)])
    plsc.subcore_barrier()
    pltpu.sync_copy(deg_acc.at[pl.ds(sid * 640, 640)],
                    deg_out.at[cid, pl.ds(sid * 640, 640)])


def _sc_prep(edge_index, efeat):
    f = pl.kernel(
        _sc_prep_body,
        out_type=[
            jax.ShapeDtypeStruct((_NC, _NP), jnp.float32),
            jax.ShapeDtypeStruct((_NW * _MWK,), jnp.int32),
        ],
        mesh=_sc_mesh(),
        scratch_types=[
            pltpu.VMEM((_AB,), jnp.int32),
            pltpu.VMEM((_AB,), jnp.int32),
            pltpu.VMEM((8, 80), jnp.int32),
            pltpu.VMEM((_AB * 3,), jnp.int32),
            pltpu.VMEM((40 * _MW,), jnp.int32),
            pltpu.VMEM((80,), jnp.float32),
            pltpu.VMEM((640,), jnp.float32),
            pltpu.VMEM_SHARED((_NP,), jnp.float32),
            pltpu.SemaphoreType.DMA,
        ],
        compiler_params=pltpu.CompilerParams(needs_layout_passes=False),
    )
    return f(edge_index[0], edge_index[1], efeat.reshape(_E * 3))


def _sc_ne_body(meta_hbm, norm_hbm, ne_out, metac, norm_v, nebuf):
    cid = lax.axis_index("c")
    sid = lax.axis_index("s")
    wid = sid * _NC + cid
    pltpu.sync_copy(meta_hbm.at[pl.ds(wid * _MWK, _MWK)], metac)
    pltpu.sync_copy(norm_hbm, norm_v)

    def blk(j, _):
        sv = metac[pl.ds(j * _MW, 16)]
        dv = metac[pl.ds(j * _MW + 16, 16)]
        nebuf[pl.ds(j * 16, 16)] = (plsc.load_gather(norm_v, [sv])
                                    * plsc.load_gather(norm_v, [dv]))
        return 0
    lax.fori_loop(0, _NBLK, blk, 0)
    pltpu.sync_copy(nebuf, ne_out.at[pl.ds(wid * _EW, _EW)])


def _sc_ne(meta, norm1):
    f = pl.kernel(
        _sc_ne_body,
        out_type=[jax.ShapeDtypeStruct((_E,), jnp.float32)],
        mesh=_sc_mesh(),
        scratch_types=[
            pltpu.VMEM((_MWK,), jnp.int32),
            pltpu.VMEM((_N,), jnp.float32),
            pltpu.VMEM((_EW,), jnp.float32),
        ],
        compiler_params=pltpu.CompilerParams(needs_layout_passes=False),
    )
    return f(meta, norm1)


def _tc_prep_kernel(nfeat_ref, atab_ref, d0_ref, d1_ref, btab_ref, w0_ref,
                    b0_ref, hp0_ref, norm_ref, rdeg_ref, ftab_ref):
    i = pl.program_id(0)
    nf = nfeat_ref[...]
    iota_av = lax.broadcasted_iota(jnp.int32, (1, _AV), 1)
    h0 = jnp.zeros((_R, _D), jnp.float32)
    for f in range(_NA):
        oh = (nf[:, f:f + 1] == iota_av).astype(jnp.float32)
        h0 = h0 + jnp.dot(oh, atab_ref[f], preferred_element_type=jnp.float32,
                          precision=lax.Precision.HIGHEST)

    dg = d0_ref[...] + d1_ref[...] + 1.0
    norm_ref[...] = lax.rsqrt(dg)
    rdeg_ref[...] = 1.0 / dg
    hp0_ref[...] = jnp.dot(h0, w0_ref[...],
                           preferred_element_type=jnp.float32) + b0_ref[...]

    @pl.when(i == 0)
    def _():
        i512 = lax.broadcasted_iota(jnp.int32, (512, 1), 0)
        iota_bv = lax.broadcasted_iota(jnp.int32, (1, _BV), 1)
        sel0 = (i512 // 64 == iota_bv).astype(jnp.float32)
        sel1 = ((i512 // 8) % 8 == iota_bv).astype(jnp.float32)
        sel2 = (i512 % 8 == iota_bv).astype(jnp.float32)
        for l in range(_L):
            ft = (jnp.dot(sel0, btab_ref[l, 0], preferred_element_type=jnp.float32,
                          precision=lax.Precision.HIGHEST)
                  + jnp.dot(sel1, btab_ref[l, 1], preferred_element_type=jnp.float32,
                            precision=lax.Precision.HIGHEST)
                  + jnp.dot(sel2, btab_ref[l, 2], preferred_element_type=jnp.float32,
                            precision=lax.Precision.HIGHEST))
            ftab_ref[l] = ft


def _tc_prep(nfeat, atom_tables, d0, d1, bond_tables, W0, b0):
    return pl.pallas_call(
        _tc_prep_kernel,
        grid=(_NRB,),
        in_specs=[
            pl.BlockSpec((_R, _NA), lambda i: (i, 0)),
            pl.BlockSpec((_NA, _AV, _D), lambda i: (0, 0, 0)),
            pl.BlockSpec((_R, 1), lambda i: (i, 0)),
            pl.BlockSpec((_R, 1), lambda i: (i, 0)),
            pl.BlockSpec((_L, _NB, _BV, _D), lambda i: (0, 0, 0, 0)),
            pl.BlockSpec((_D, _D), lambda i: (0, 0)),
            pl.BlockSpec((1, _D), lambda i: (0, 0)),
        ],
        out_specs=[
            pl.BlockSpec((_R, _D), lambda i: (i, 0)),
            pl.BlockSpec((_R, 1), lambda i: (i, 0)),
            pl.BlockSpec((_R, 1), lambda i: (i, 0)),
            pl.BlockSpec((_L, 512, _D), lambda i: (0, 0, 0)),
        ],
        out_shape=[
            jax.ShapeDtypeStruct((_N, _D), jnp.float32),
            jax.ShapeDtypeStruct((_N, 1), jnp.float32),
            jax.ShapeDtypeStruct((_N, 1), jnp.float32),
            jax.ShapeDtypeStruct((_L, 512, _D), jnp.float32),
        ],
    )(nfeat, atom_tables, d0, d1, bond_tables, W0, b0)


def _sc_edge_body(hp_hbm, meta_hbm, ne_hbm, ftab_hbm, agg_out,
                  efbuf, metab, srcb, dstb, fusb, neb, buf, agg_acc,
                  msem0, msem1, msem2, msem3,
                  gsem0, gsem1, gsem2, gsem3,
                  ssem0, ssem1, ssem2, ssem3, zsem):
    cid = lax.axis_index("c")
    sid = lax.axis_index("s")
    wid = sid * _NC + cid
    msems = (msem0, msem1, msem2, msem3)
    gsems = (gsem0, gsem1, gsem2, gsem3)
    ssems = (ssem0, ssem1, ssem2, ssem3)
    mbase = wid * _MWK
    ebase = wid * _EW

    for t in range(4):
        def _z(k, _, t=t):
            for c in range(8):
                buf[t, k, pl.ds(c * 16, 16)] = jnp.zeros((16,), jnp.float32)
            return 0
        lax.fori_loop(0, _B, _z, 0)
    z0 = sid * 640

    @pl.when(sid < 15)
    def _():
        for m in range(40):
            pltpu.async_copy(buf.at[0], agg_acc.at[pl.ds(z0 + m * 16, 16), :],
                             zsem)
        for m in range(40):
            pltpu.make_async_copy(buf.at[0], agg_acc.at[pl.ds(z0, 16), :],
                                  zsem).wait()

    @pl.when(sid == 15)
    def _():
        for m in range(25):
            pltpu.async_copy(buf.at[0], agg_acc.at[pl.ds(9600 + m * 16, 16), :],
                             zsem)
        for m in range(25):
            pltpu.make_async_copy(buf.at[0], agg_acc.at[pl.ds(9600, 16), :],
                                  zsem).wait()

    plsc.subcore_barrier()

    def start_meta(j, t):
        pltpu.async_copy(meta_hbm.at[pl.ds(mbase + j * _MW, _MW)],
                         metab.at[t], msems[t])

    def wait_meta(t):
        pltpu.make_async_copy(meta_hbm.at[pl.ds(0, _MW)], metab.at[t],
                              msems[t]).wait()

    def stage(t):
        srcb[t, pl.ds(0, 16)] = metab[t, pl.ds(0, 16)]
        dstb[t, pl.ds(0, 16)] = metab[t, pl.ds(16, 16)]
        fusb[t, pl.ds(0, 16)] = metab[t, pl.ds(32, 16)]

    def start_gather(j, t):
        pltpu.async_copy(hp_hbm.at[srcb.at[t]], buf.at[t], gsems[t])
        pltpu.async_copy(ftab_hbm.at[fusb.at[t]], efbuf.at[t], gsems[t])
        pltpu.async_copy(ne_hbm.at[pl.ds(ebase + j * 16, 16)], neb.at[t],
                         gsems[t])

    def wait_gather(t):
        pltpu.make_async_copy(hp_hbm.at[srcb.at[t]], buf.at[t],
                              gsems[t]).wait()
        pltpu.make_async_copy(ftab_hbm.at[fusb.at[t]], efbuf.at[t],
                              gsems[t]).wait()
        pltpu.make_async_copy(ne_hbm.at[pl.ds(0, 16)], neb.at[t],
                              gsems[t]).wait()

    def start_scatter(t):
        pltpu.async_copy(buf.at[t], agg_acc.at[dstb.at[t]], ssems[t],
                         add=True)

    def wait_scatter(t):
        pltpu.make_async_copy(buf.at[t], agg_acc.at[dstb.at[t]],
                              ssems[t]).wait()

    def compute(t):
        ne16 = neb[t, pl.ds(0, 16)]
        for ii in range(16):
            nev = jnp.full((16,), ne16[ii], jnp.float32)
            for c in range(8):
                v = buf[t, ii, pl.ds(c * 16, 16)]
                tv = efbuf[t, ii, pl.ds(c * 16, 16)]
                buf[t, ii, pl.ds(c * 16, 16)] = (
                    jnp.maximum(v + tv, 0.0) * nev)

    for t in range(4):
        start_meta(t, t)
    for jj in range(2):
        wait_meta(jj)
        stage(jj)
        start_gather(jj, jj)

    def quad(q, _):
        for s in range(4):
            j = 4 * q + s
            s2 = (s + 2) % 4

            @pl.when(j + 2 < _NBLK)
            def _():
                wait_meta(s2)

            @pl.when(j >= 2)
            def _():
                wait_scatter(s2)

            @pl.when(j + 2 < _NBLK)
            def _():
                stage(s2)
                start_gather(j + 2, s2)

            @pl.when(j + 4 < _NBLK)
            def _():
                start_meta(j + 4, s)

            wait_gather(s)
            compute(s)
            start_scatter(s)
        return 0

    lax.fori_loop(0, _NBLK // 4, quad, 0)
    wait_gather(0)
    compute(0)
    start_scatter(0)
    wait_scatter(2)
    wait_scatter(3)
    wait_scatter(0)
    plsc.subcore_barrier()

    @pl.when(sid < 15)
    def _():
        pltpu.sync_copy(agg_acc.at[pl.ds(z0, 640), :],
                        agg_out.at[cid, pl.ds(z0, 640), :])

    @pl.when(sid == 15)
    def _():
        pltpu.sync_copy(agg_acc.at[pl.ds(9600, 400), :],
                        agg_out.at[cid, pl.ds(9600, 400), :])


def _sc_edge_make():
    return pl.kernel(
        _sc_edge_body,
        out_type=[jax.ShapeDtypeStruct((_NC, _N, _D), jnp.float32)],
        mesh=_sc_mesh(),
        scratch_types=[
            pltpu.VMEM((4, _B, _D), jnp.float32),
            pltpu.VMEM((4, _MW), jnp.int32),
            pltpu.VMEM((4, 16), jnp.int32),
            pltpu.VMEM((4, 16), jnp.int32),
            pltpu.VMEM((4, 16), jnp.int32),
            pltpu.VMEM((4, 16), jnp.float32),
            pltpu.VMEM((4, _B, _D), jnp.float32),
            pltpu.VMEM_SHARED((_N, _D), jnp.float32),
        ] + [pltpu.SemaphoreType.DMA] * 13,
        compiler_params=pltpu.CompilerParams(needs_layout_passes=False),
    )


def _tc_dense_kernel(last, agg0_ref, agg1_ref, hp_ref, rdeg_ref, res_ref,
                     gam_ref, bet_ref, wn_ref, bn_ref,
                     out_ref, hpre, stats):
    p = pl.program_id(0)
    i = pl.program_id(1)

    @pl.when(p == 0)
    def _():
        resid = jnp.maximum(hp_ref[...] + res_ref[...], 0.0) * rdeg_ref[...]
        pre = agg0_ref[...] + agg1_ref[...] + resid
        hpre[pl.ds(i * _R, _R), :] = pre
        s = jnp.sum(pre, axis=0, keepdims=True)
        sq = jnp.sum(pre * pre, axis=0, keepdims=True)

        @pl.when(i == 0)
        def _():
            stats[0:1, :] = s
            stats[1:2, :] = sq

        @pl.when(i > 0)
        def _():
            stats[0:1, :] = stats[0:1, :] + s
            stats[1:2, :] = stats[1:2, :] + sq

    @pl.when(p == 1)
    def _():
        mean = stats[0:1, :] / float(_N)
        var = stats[1:2, :] / float(_N) - mean * mean
        inv = lax.rsqrt(var + 1e-5)
        h = jnp.maximum((hpre[pl.ds(i * _R, _R), :] - mean) * inv * gam_ref[...]
                        + bet_ref[...], 0.0)
        if last:
            out_ref[...] = h
        else:
            out_ref[...] = jnp.dot(h, wn_ref[...],
                                   preferred_element_type=jnp.float32) + bn_ref[...]


def _tc_dense(agg0, agg1, hp, rdeg_c, res_l, gam_l, bet_l, wn, bn, last):
    return pl.pallas_call(
        functools.partial(_tc_dense_kernel, last),
        grid=(2, _NRB),
        in_specs=[
            pl.BlockSpec((_R, _D), lambda p, i: (i, 0)),
            pl.BlockSpec((_R, _D), lambda p, i: (i, 0)),
            pl.BlockSpec((_R, _D), lambda p, i: (i, 0)),
            pl.BlockSpec((_R, 1), lambda p, i: (i, 0)),
            pl.BlockSpec((1, _D), lambda p, i: (0, 0)),
            pl.BlockSpec((1, _D), lambda p, i: (0, 0)),
            pl.BlockSpec((1, _D), lambda p, i: (0, 0)),
            pl.BlockSpec((_D, _D), lambda p, i: (0, 0)),
            pl.BlockSpec((1, _D), lambda p, i: (0, 0)),
        ],
        out_specs=pl.BlockSpec((_R, _D), lambda p, i: (i, 0)),
        out_shape=jax.ShapeDtypeStruct((_N, _D), jnp.float32),
        scratch_shapes=[pltpu.VMEM((_N, _D), jnp.float32),
                        pltpu.VMEM((8, _D), jnp.float32)],
    )(agg0, agg1, hp, rdeg_c, res_l, gam_l, bet_l, wn, bn)


def _tc_pool_kernel(h_ref, gid_ref, w1_ref, b1_ref, w2_ref, b2_ref,
                    out_ref, gsum, cnt):
    i = pl.program_id(0)
    ids = gid_ref[...]
    iota_g = lax.broadcasted_iota(jnp.int32, (1, _G), 1)
    oh = (ids == iota_g).astype(jnp.float32)
    cdims = (((0,), (0,)), ((), ()))
    part = lax.dot_general(oh, h_ref[...], dimension_numbers=cdims,
                           preferred_element_type=jnp.float32,
                           precision=lax.Precision.HIGHEST)
    c = lax.dot_general(oh, jnp.ones((_R, 1), jnp.float32),
                        dimension_numbers=cdims,
                        preferred_element_type=jnp.float32,
                        precision=lax.Precision.HIGHEST)

    @pl.when(i == 0)
    def _():
        gsum[...] = part
        cnt[:, 0:1] = c

    @pl.when(i > 0)
    def _():
        gsum[...] = gsum[...] + part
        cnt[:, 0:1] = cnt[:, 0:1] + c

    @pl.when(i == _NRB - 1)
    def _():
        g = gsum[...] / jnp.maximum(cnt[:, 0:1], 1.0)
        z = jnp.maximum(jnp.dot(g, w1_ref[...],
                                preferred_element_type=jnp.float32)
                        + b1_ref[...], 0.0)
        out_ref[...] = jnp.dot(z, w2_ref[...],
                               preferred_element_type=jnp.float32) + b2_ref[...]


def _tc_pool(h, gid2d, W1, b1, W2, b2):
    return pl.pallas_call(
        _tc_pool_kernel,
        grid=(_NRB,),
        in_specs=[
            pl.BlockSpec((_R, _D), lambda i: (i, 0)),
            pl.BlockSpec((_R, 1), lambda i: (i, 0)),
            pl.BlockSpec((_D, _D // 2), lambda i: (0, 0)),
            pl.BlockSpec((1, _D // 2), lambda i: (0, 0)),
            pl.BlockSpec((_D // 2, 1), lambda i: (0, 0)),
            pl.BlockSpec((1, 1), lambda i: (0, 0)),
        ],
        out_specs=pl.BlockSpec((_G, 1), lambda i: (0, 0)),
        out_shape=jax.ShapeDtypeStruct((_G, 1), jnp.float32),
        scratch_shapes=[
            pltpu.VMEM((_G, _D), jnp.float32),
            pltpu.VMEM((_G, 8), jnp.float32),
        ],
    )(h, gid2d, W1, b1, W2, b2)


def kernel(nfeat, efeat, edge_index, node_graph_ids, atom_tables, bond_tables,
           W, b, res_w, gamma, beta, W1, b1, W2, b2):
    deg_parts, meta = _sc_prep(edge_index, efeat)

    d0 = deg_parts[0, :_N].reshape(_N, 1)
    d1 = deg_parts[1, :_N].reshape(_N, 1)
    hp, norm2, rdeg2, ftabs = _tc_prep(
        nfeat, atom_tables, d0, d1, bond_tables,
        W[0], b[0].reshape(1, _D))

    norm1 = norm2.reshape(_N)
    rdeg_c = rdeg2
    (ne,) = _sc_ne(meta, norm1)

    sc_edge = _sc_edge_make()
    for l in range(_L):
        (aggp,) = sc_edge(hp, meta, ne, ftabs[l])
        last = l == _L - 1
        wn = W[l] if last else W[l + 1]
        bn = (b[l] if last else b[l + 1]).reshape(1, _D)
        hp = _tc_dense(aggp[0], aggp[1], hp, rdeg_c,
                       res_w[l], gamma[l].reshape(1, _D),
                       beta[l].reshape(1, _D), wn, bn, last)

    return _tc_pool(hp, node_graph_ids.reshape(_N, 1),
                    W1, b1.reshape(1, _D // 2), W2, b2.reshape(1, 1))
